# Initial kernel scaffold; baseline (speedup 1.0000x reference)
import jax
import jax.numpy as jnp
from jax.experimental import pallas as pl


def kernel(x, edge_index, weight):
    raise NotImplementedError("write your pallas kernel here")


# speedup vs baseline: 74.1468x; 74.1468x over previous
import functools

import jax
import jax.numpy as jnp
from jax import lax
from jax.experimental import pallas as pl
from jax.experimental.pallas import tpu as pltpu
from jax.experimental.pallas import tpu_sc as plsc

_N = 100000
_C = 3
_E = 6400000
_K = 50000

_R = 1024
_L = 128
_SZ = _R * _L

_NC = 2
_NS = 16
_NW = _NC * _NS

_PB = 1664
_KP = _NW * _PB
_PCH = _KP // 16
_NMP = 100352
_ECH = 2000
_EPW = _E // _NW
_NECH = _EPW // _ECH


def _cmp_exchange(key, idx, j, k, row_i, lane_i):
    if j < _R:
        g = _R // (2 * j)
        def _sw(a):
            a4 = a.reshape(g, 2, j, _L)
            return jnp.concatenate([a4[:, 1:2], a4[:, 0:1]], axis=1).reshape(_R, _L)
        kp, ip = _sw(key), _sw(idx)
        is_lo = (row_i & j) == 0
    else:
        jc = j // _R
        def _shl(a):
            return jnp.concatenate([a[:, jc:], a[:, :jc]], axis=1)
        def _shr(a):
            return jnp.concatenate([a[:, _L - jc:], a[:, :_L - jc]], axis=1)
        is_lo = (lane_i & jc) == 0
        kp = jnp.where(is_lo, _shl(key), _shr(key))
        ip = jnp.where(is_lo, _shl(idx), _shr(idx))

    if k < _R:
        desc = (row_i & k) == 0
        keep_better = is_lo == desc
    elif k < _SZ:
        desc = (lane_i & (k // _R)) == 0
        keep_better = is_lo == desc
    else:
        keep_better = is_lo

    mine_better = (key > kp) | ((key == kp) & (idx < ip))
    take_mine = mine_better == keep_better
    return jnp.where(take_mine, key, kp), jnp.where(take_mine, idx, ip)


def _sort_body(s_ref, ks_ref, is_ref):
    row_i = lax.broadcasted_iota(jnp.int32, (_R, _L), 0)
    lane_i = lax.broadcasted_iota(jnp.int32, (_R, _L), 1)
    key = s_ref[...]
    idx = lane_i * _R + row_i
    for m in range(1, 18):
        for je in range(m - 1, -1, -1):
            key, idx = _cmp_exchange(key, idx, 1 << je, 1 << m, row_i, lane_i)
    ks_ref[...] = key
    is_ref[...] = idx


_score_sort = pl.pallas_call(
    _sort_body,
    out_shape=(
        jax.ShapeDtypeStruct((_R, _L), jnp.float32),
        jax.ShapeDtypeStruct((_R, _L), jnp.int32),
    ),
    in_specs=[
        pl.BlockSpec(memory_space=pltpu.VMEM),
    ],
    out_specs=(
        pl.BlockSpec(memory_space=pltpu.VMEM),
        pl.BlockSpec(memory_space=pltpu.VMEM),
    ),
)


def _sc_body(edge_hbm, perm_hbm, score_hbm, xflat_hbm,
             eout_hbm, xo_hbm,
             node_map, pbuf, ps, sbuf, pidx2, gtmp,
             es_in, ed_in, es_out, ed_out, sem):
    wid = lax.axis_index("s") * _NC + lax.axis_index("c")

    neg1 = jnp.full((16,), -1, jnp.int32)
    def _mset(i, _):
        node_map[pl.ds(i * 16, 16)] = neg1
        return 0
    lax.fori_loop(0, _NMP // 16, _mset, 0)

    lane = lax.iota(jnp.int32, 16)
    for ch in range(_KP // _PCH):
        base = ch * _PCH
        valid = min(_PCH, max(0, _K - base))
        if valid == 0:
            break
        pltpu.sync_copy(perm_hbm.at[pl.ds(base, _PCH)], pbuf)
        def _scat(i, _, base=base):
            tgt = pbuf[pl.ds(i * 16, 16)]
            vals = base + i * 16 + lane
            plsc.store_scatter(node_map, [tgt], vals)
            return 0
        lax.fori_loop(0, valid // 16, _scat, 0)

    pbase = wid * _PB
    pltpu.sync_copy(perm_hbm.at[pl.ds(pbase, _PB)], ps)
    pltpu.sync_copy(score_hbm.at[pl.ds(pbase, _PB)], sbuf)
    for c in range(_C):
        for sub in range(_PB // 128):
            def _mkidx(i, _, c=c, sub=sub):
                p = ps[pl.ds(sub * 128 + i * 16, 16)]
                pidx2[sub, pl.ds(i * 16, 16)] = p * _C + c
                return 0
            lax.fori_loop(0, 8, _mkidx, 0)
            pltpu.async_copy(xflat_hbm.at[pidx2.at[sub]], gtmp, sem).wait()
            def _mul(i, _, sub=sub):
                gtmp[pl.ds(i * 16, 16)] = (
                    gtmp[pl.ds(i * 16, 16)] * sbuf[pl.ds(sub * 128 + i * 16, 16)])
                return 0
            lax.fori_loop(0, 8, _mul, 0)
            pltpu.sync_copy(
                gtmp, xo_hbm.at[pl.ds(c * _KP + pbase + sub * 128, 128)])

    ebase = wid * _EPW
    def _echunk(g, _):
        off = ebase + g * _ECH
        pltpu.sync_copy(edge_hbm.at[pl.ds(off, _ECH)], es_in)
        pltpu.sync_copy(edge_hbm.at[pl.ds(_E + off, _ECH)], ed_in)
        def _ebody(i, _):
            sl = pl.ds(i * 16, 16)
            rs = plsc.load_gather(node_map, [es_in[sl]])
            cs = plsc.load_gather(node_map, [ed_in[sl]])
            keep = (rs >= 0) & (cs >= 0)
            es_out[sl] = jnp.where(keep, rs, -1)
            ed_out[sl] = jnp.where(keep, cs, -1)
            return 0
        lax.fori_loop(0, _ECH // 16, _ebody, 0)
        pltpu.sync_copy(es_out, eout_hbm.at[pl.ds(off, _ECH)])
        pltpu.sync_copy(ed_out, eout_hbm.at[pl.ds(_E + off, _ECH)])
        return 0
    lax.fori_loop(0, _NECH, _echunk, 0)


def _make_sc_remap():
    return functools.partial(
        pl.kernel,
    out_type=(
        jax.ShapeDtypeStruct((2 * _E,), jnp.int32),
        jax.ShapeDtypeStruct((_C * _KP,), jnp.float32),
    ),
    mesh=plsc.VectorSubcoreMesh(
        core_axis_name="c", subcore_axis_name="s",
        num_cores=_NC, num_subcores=_NS),
    compiler_params=pltpu.CompilerParams(needs_layout_passes=False),
    scratch_types=[
        pltpu.VMEM((_NMP,), jnp.int32),
        pltpu.VMEM((_PCH,), jnp.int32),
        pltpu.VMEM((_PB,), jnp.int32),
        pltpu.VMEM((_PB,), jnp.float32),
        pltpu.VMEM((_PB // 128, 128), jnp.int32),
        pltpu.VMEM((128,), jnp.float32),
        pltpu.VMEM((_ECH,), jnp.int32),
        pltpu.VMEM((_ECH,), jnp.int32),
        pltpu.VMEM((_ECH,), jnp.int32),
        pltpu.VMEM((_ECH,), jnp.int32),
        pltpu.SemaphoreType.DMA,
    ],
    )(_sc_body)


def kernel(x, edge_index, weight):
    score = jnp.tanh(
        jnp.sum(x * weight[None, :], axis=-1)
        / (jnp.linalg.norm(weight) + 1e-16))
    spad = jnp.pad(score, (0, _SZ - _N), constant_values=-jnp.inf)
    sarr = spad.reshape(_L, _R).T
    ks, iz = _score_sort(sarr)
    score_sorted = ks.swapaxes(0, 1).reshape(_SZ)
    perm_sorted = iz.swapaxes(0, 1).reshape(_SZ)
    perm = perm_sorted[:_K]
    score_sel = score_sorted[:_K]

    perm_pad = jnp.pad(perm, (0, _KP - _K))
    score_pad = jnp.pad(score_sel, (0, _KP - _K))
    eout, xo = _make_sc_remap()(
        edge_index.reshape(-1), perm_pad, score_pad, x.reshape(-1))
    x_out = xo.reshape(_C, _KP)[:, :_K].T
    return x_out, eout.reshape(2, _E), perm


# speedup vs baseline: 82.9436x; 1.1186x over previous
import functools

import jax
import jax.numpy as jnp
from jax import lax
from jax.experimental import pallas as pl
from jax.experimental.pallas import tpu as pltpu
from jax.experimental.pallas import tpu_sc as plsc

_N = 100000
_C = 3
_E = 6400000
_K = 50000

_R = 1024
_L = 128
_SZ = _R * _L

_NC = 2
_NS = 16
_NW = _NC * _NS

_PB = 1664
_KP = _NW * _PB
_PCH = 2000
_NMP = 100352
_ECH = 2000
_EPW = _E // _NW
_NECH = _EPW // _ECH


def _cmp_exchange(key, idx, j, k, row_i, lane_i):
    if j < _R:
        g = _R // (2 * j)
        def _sw(a):
            a4 = a.reshape(g, 2, j, _L)
            return jnp.concatenate([a4[:, 1:2], a4[:, 0:1]], axis=1).reshape(_R, _L)
        kp, ip = _sw(key), _sw(idx)
        is_lo = (row_i & j) == 0
    else:
        jc = j // _R
        def _shl(a):
            return jnp.concatenate([a[:, jc:], a[:, :jc]], axis=1)
        def _shr(a):
            return jnp.concatenate([a[:, _L - jc:], a[:, :_L - jc]], axis=1)
        is_lo = (lane_i & jc) == 0
        kp = jnp.where(is_lo, _shl(key), _shr(key))
        ip = jnp.where(is_lo, _shl(idx), _shr(idx))

    if k < _R:
        desc = (row_i & k) == 0
        keep_better = is_lo == desc
    elif k < _SZ:
        desc = (lane_i & (k // _R)) == 0
        keep_better = is_lo == desc
    else:
        keep_better = is_lo

    mine_better = (key > kp) | ((key == kp) & (idx < ip))
    take_mine = mine_better == keep_better
    return jnp.where(take_mine, key, kp), jnp.where(take_mine, idx, ip)


def _sort_body(s_ref, ks_ref, is_ref):
    row_i = lax.broadcasted_iota(jnp.int32, (_R, _L), 0)
    lane_i = lax.broadcasted_iota(jnp.int32, (_R, _L), 1)
    key = s_ref[...]
    idx = lane_i * _R + row_i
    for m in range(1, 18):
        for je in range(m - 1, -1, -1):
            key, idx = _cmp_exchange(key, idx, 1 << je, 1 << m, row_i, lane_i)
    ks_ref[...] = key
    is_ref[...] = idx


_score_sort = pl.pallas_call(
    _sort_body,
    out_shape=(
        jax.ShapeDtypeStruct((_R, _L), jnp.float32),
        jax.ShapeDtypeStruct((_R, _L), jnp.int32),
    ),
    in_specs=[
        pl.BlockSpec(memory_space=pltpu.VMEM),
    ],
    out_specs=(
        pl.BlockSpec(memory_space=pltpu.VMEM),
        pl.BlockSpec(memory_space=pltpu.VMEM),
    ),
)


def _sc_body(edge_hbm, perm_hbm, score_hbm, xflat_hbm,
             eout_hbm, xo_hbm,
             node_map, ps, sbuf, pidx2, gbuf,
             ei0s, ei0d, ei1s, ei1d, eo0s, eo0d, eo1s, eo1d,
             sin0, sin1, sout0, sout1, sgat):
    wid = lax.axis_index("s") * _NC + lax.axis_index("c")
    pbase = wid * _PB
    ebase = wid * _EPW
    ein = ((ei0s, ei0d), (ei1s, ei1d))
    eout_b = ((eo0s, eo0d), (eo1s, eo1d))
    sin = (sin0, sin1)
    sout = (sout0, sout1)
    lane = lax.iota(jnp.int32, 16)
    neg1 = jnp.full((16,), -1, jnp.int32)

    pltpu.async_copy(perm_hbm.at[pl.ds(0, _PCH)], ei0s, sin0)
    pltpu.async_copy(perm_hbm.at[pl.ds(pbase, _PB)], ps, sgat)
    pltpu.async_copy(score_hbm.at[pl.ds(pbase, _PB)], sbuf, sgat)

    def _mset(t, _):
        for u in range(8):
            node_map[pl.ds(t * 128 + u * 16, 16)] = neg1
        return 0
    lax.fori_loop(0, _NMP // 128, _mset, 0)

    nch = _K // _PCH
    for ch in range(nch):
        b = ch & 1
        base = ch * _PCH
        pltpu.make_async_copy(
            perm_hbm.at[pl.ds(base, _PCH)], ein[b][0], sin[b]).wait()
        if ch + 1 < nch:
            pltpu.async_copy(
                perm_hbm.at[pl.ds(base + _PCH, _PCH)], ein[1 - b][0],
                sin[1 - b])
        def _scat(t, _, b=b, base=base):
            for u in range(5):
                o = t * 80 + u * 16
                tgt = ein[b][0][pl.ds(o, 16)]
                plsc.store_scatter(node_map, [tgt], base + o + lane)
            return 0
        lax.fori_loop(0, _PCH // 80, _scat, 0)

    pltpu.make_async_copy(perm_hbm.at[pl.ds(pbase, _PB)], ps, sgat).wait()
    pltpu.make_async_copy(score_hbm.at[pl.ds(pbase, _PB)], sbuf, sgat).wait()
    nsub = _PB // 128
    for c in range(_C):
        for sub in range(nsub):
            for u in range(8):
                pidx2[sub, pl.ds(u * 16, 16)] = (
                    ps[pl.ds(sub * 128 + u * 16, 16)] * _C + c)
            pltpu.async_copy(
                xflat_hbm.at[pidx2.at[sub]],
                gbuf.at[pl.ds(sub * 128, 128)], sgat)
        for sub in range(nsub):
            pltpu.make_async_copy(
                xflat_hbm.at[pidx2.at[sub]],
                gbuf.at[pl.ds(sub * 128, 128)], sgat).wait()
        def _mul(t, _):
            for u in range(4):
                sl = pl.ds(t * 64 + u * 16, 16)
                gbuf[sl] = gbuf[sl] * sbuf[sl]
            return 0
        lax.fori_loop(0, _PB // 64, _mul, 0)
        pltpu.sync_copy(gbuf, xo_hbm.at[pl.ds(c * _KP + pbase, _PB)])

    def _start_in(b, off):
        pltpu.async_copy(edge_hbm.at[pl.ds(off, _ECH)], ein[b][0], sin[b])
        pltpu.async_copy(edge_hbm.at[pl.ds(_E + off, _ECH)], ein[b][1], sin[b])
    def _wait_in(b, off):
        pltpu.make_async_copy(
            edge_hbm.at[pl.ds(off, _ECH)], ein[b][0], sin[b]).wait()
        pltpu.make_async_copy(
            edge_hbm.at[pl.ds(_E + off, _ECH)], ein[b][1], sin[b]).wait()
    def _start_out(b, off):
        pltpu.async_copy(eout_b[b][0], eout_hbm.at[pl.ds(off, _ECH)], sout[b])
        pltpu.async_copy(
            eout_b[b][1], eout_hbm.at[pl.ds(_E + off, _ECH)], sout[b])
    def _wait_out(b, off):
        pltpu.make_async_copy(
            eout_b[b][0], eout_hbm.at[pl.ds(off, _ECH)], sout[b]).wait()
        pltpu.make_async_copy(
            eout_b[b][1], eout_hbm.at[pl.ds(_E + off, _ECH)], sout[b]).wait()

    _start_in(0, ebase)
    _start_in(1, ebase + _ECH)

    def _epair(i, _):
        for b in range(2):
            off = ebase + (2 * i + b) * _ECH
            _wait_in(b, off)

            @pl.when(i > 0)
            def _():
                _wait_out(b, off - 2 * _ECH)

            def _ebody(t, _, b=b):
                for u in range(5):
                    sl = pl.ds(t * 80 + u * 16, 16)
                    rs = plsc.load_gather(node_map, [ein[b][0][sl]])
                    cs = plsc.load_gather(node_map, [ein[b][1][sl]])
                    keep = (rs >= 0) & (cs >= 0)
                    eout_b[b][0][sl] = jnp.where(keep, rs, -1)
                    eout_b[b][1][sl] = jnp.where(keep, cs, -1)
                return 0
            lax.fori_loop(0, _ECH // 80, _ebody, 0)
            _start_out(b, off)

            @pl.when(i < _NECH // 2 - 1)
            def _():
                _start_in(b, off + 2 * _ECH)
        return 0
    lax.fori_loop(0, _NECH // 2, _epair, 0)
    _wait_out(0, ebase + (_NECH - 2) * _ECH)
    _wait_out(1, ebase + (_NECH - 1) * _ECH)


def _make_sc_remap():
    return functools.partial(
        pl.kernel,
    out_type=(
        jax.ShapeDtypeStruct((2 * _E,), jnp.int32),
        jax.ShapeDtypeStruct((_C * _KP,), jnp.float32),
    ),
    mesh=plsc.VectorSubcoreMesh(
        core_axis_name="c", subcore_axis_name="s",
        num_cores=_NC, num_subcores=_NS),
    compiler_params=pltpu.CompilerParams(needs_layout_passes=False),
    scratch_types=[
        pltpu.VMEM((_NMP,), jnp.int32),
        pltpu.VMEM((_PB,), jnp.int32),
        pltpu.VMEM((_PB,), jnp.float32),
        pltpu.VMEM((_PB // 128, 128), jnp.int32),
        pltpu.VMEM((_PB,), jnp.float32),
        pltpu.VMEM((_ECH,), jnp.int32),
        pltpu.VMEM((_ECH,), jnp.int32),
        pltpu.VMEM((_ECH,), jnp.int32),
        pltpu.VMEM((_ECH,), jnp.int32),
        pltpu.VMEM((_ECH,), jnp.int32),
        pltpu.VMEM((_ECH,), jnp.int32),
        pltpu.VMEM((_ECH,), jnp.int32),
        pltpu.VMEM((_ECH,), jnp.int32),
        pltpu.SemaphoreType.DMA,
        pltpu.SemaphoreType.DMA,
        pltpu.SemaphoreType.DMA,
        pltpu.SemaphoreType.DMA,
        pltpu.SemaphoreType.DMA,
    ],
    )(_sc_body)


def kernel(x, edge_index, weight):
    score = jnp.tanh(
        jnp.sum(x * weight[None, :], axis=-1)
        / (jnp.linalg.norm(weight) + 1e-16))
    spad = jnp.pad(score, (0, _SZ - _N), constant_values=-jnp.inf)
    sarr = spad.reshape(_L, _R).T
    ks, iz = _score_sort(sarr)
    score_sorted = ks.swapaxes(0, 1).reshape(_SZ)
    perm_sorted = iz.swapaxes(0, 1).reshape(_SZ)
    perm = perm_sorted[:_K]
    score_sel = score_sorted[:_K]

    perm_pad = jnp.pad(perm, (0, _KP - _K))
    score_pad = jnp.pad(score_sel, (0, _KP - _K))
    eout, xo = _make_sc_remap()(
        edge_index.reshape(-1), perm_pad, score_pad, x.reshape(-1))
    x_out = xo.reshape(_C, _KP)[:, :_K].T
    return x_out, eout.reshape(2, _E), perm


# speedup vs baseline: 359.6919x; 4.3366x over previous
import functools

import jax
import jax.numpy as jnp
from jax import lax
from jax.experimental import pallas as pl
from jax.experimental.pallas import tpu as pltpu
from jax.experimental.pallas import tpu_sc as plsc

_N = 100000
_C = 3
_E = 6400000
_K = 50000

_R = 1024
_L = 128
_SZ = _R * _L

_NC = 2
_NS = 16
_NW = _NC * _NS

_PB = 1664
_KP = _NW * _PB
_PCH = 2000
_NMP = 100352
_ECH = 2048
_NCHG = _E // _ECH
_SLOTS = 98


def _cmp_exchange(key, idx, j, k, row_i, lane_i):
    if j < _R:
        g = _R // (2 * j)
        def _sw(a):
            a4 = a.reshape(g, 2, j, _L)
            return jnp.concatenate([a4[:, 1:2], a4[:, 0:1]], axis=1).reshape(_R, _L)
        kp, ip = _sw(key), _sw(idx)
        is_lo = (row_i & j) == 0
    else:
        jc = j // _R
        def _shl(a):
            return jnp.concatenate([a[:, jc:], a[:, :jc]], axis=1)
        def _shr(a):
            return jnp.concatenate([a[:, _L - jc:], a[:, :_L - jc]], axis=1)
        is_lo = (lane_i & jc) == 0
        kp = jnp.where(is_lo, _shl(key), _shr(key))
        ip = jnp.where(is_lo, _shl(idx), _shr(idx))

    if k < _R:
        desc = (row_i & k) == 0
        keep_better = is_lo == desc
    elif k < _SZ:
        desc = (lane_i & (k // _R)) == 0
        keep_better = is_lo == desc
    else:
        keep_better = is_lo

    mine_better = (key > kp) | ((key == kp) & (idx < ip))
    take_mine = mine_better == keep_better
    return jnp.where(take_mine, key, kp), jnp.where(take_mine, idx, ip)


def _sort_body(s_ref, ks_ref, is_ref):
    row_i = lax.broadcasted_iota(jnp.int32, (_R, _L), 0)
    lane_i = lax.broadcasted_iota(jnp.int32, (_R, _L), 1)
    key = s_ref[...]
    idx = lane_i * _R + row_i
    for m in range(1, 18):
        for je in range(m - 1, -1, -1):
            key, idx = _cmp_exchange(key, idx, 1 << je, 1 << m, row_i, lane_i)
    ks_ref[...] = key
    is_ref[...] = idx


_score_sort = pl.pallas_call(
    _sort_body,
    out_shape=(
        jax.ShapeDtypeStruct((_R, _L), jnp.float32),
        jax.ShapeDtypeStruct((_R, _L), jnp.int32),
    ),
    in_specs=[
        pl.BlockSpec(memory_space=pltpu.VMEM),
    ],
    out_specs=(
        pl.BlockSpec(memory_space=pltpu.VMEM),
        pl.BlockSpec(memory_space=pltpu.VMEM),
    ),
)


def _sc_body(edge_hbm, perm_hbm, score_hbm, xflat_hbm,
             eout_hbm, xo_hbm,
             node_map, ps, sbuf, pidx2, gbuf, pb0, pb1,
             ei0, ei1, eo0, eo1,
             sin0, sin1, sout0, sout1, sgat):
    wid = lax.axis_index("s") * _NC + lax.axis_index("c")
    pbase = wid * _PB
    ein = (ei0, ei1)
    eout_b = (eo0, eo1)
    sin = (sin0, sin1)
    sout = (sout0, sout1)
    lane = lax.iota(jnp.int32, 16)
    neg1 = jnp.full((16,), -1, jnp.int32)

    pb = (pb0, pb1)

    pltpu.async_copy(perm_hbm.at[pl.ds(0, _PCH)], pb0, sin0)
    pltpu.async_copy(perm_hbm.at[pl.ds(pbase, _PB)], ps, sgat)
    pltpu.async_copy(score_hbm.at[pl.ds(pbase, _PB)], sbuf, sgat)

    def _mset(t, _):
        for u in range(8):
            node_map[pl.ds(t * 128 + u * 16, 16)] = neg1
        return 0
    lax.fori_loop(0, _NMP // 128, _mset, 0)

    nch = _K // _PCH
    for ch in range(nch):
        b = ch & 1
        base = ch * _PCH
        pltpu.make_async_copy(
            perm_hbm.at[pl.ds(base, _PCH)], pb[b], sin[b]).wait()
        if ch + 1 < nch:
            pltpu.async_copy(
                perm_hbm.at[pl.ds(base + _PCH, _PCH)], pb[1 - b],
                sin[1 - b])
        def _scat(t, _, b=b, base=base):
            for u in range(5):
                o = t * 80 + u * 16
                tgt = pb[b][pl.ds(o, 16)]
                plsc.store_scatter(node_map, [tgt], base + o + lane)
            return 0
        lax.fori_loop(0, _PCH // 80, _scat, 0)

    pltpu.make_async_copy(perm_hbm.at[pl.ds(pbase, _PB)], ps, sgat).wait()
    pltpu.make_async_copy(score_hbm.at[pl.ds(pbase, _PB)], sbuf, sgat).wait()
    nsub = _PB // 128
    for c in range(_C):
        for sub in range(nsub):
            for u in range(8):
                pidx2[sub, pl.ds(u * 16, 16)] = (
                    ps[pl.ds(sub * 128 + u * 16, 16)] * _C + c)
            pltpu.async_copy(
                xflat_hbm.at[pidx2.at[sub]],
                gbuf.at[pl.ds(sub * 128, 128)], sgat)
        for sub in range(nsub):
            pltpu.make_async_copy(
                xflat_hbm.at[pidx2.at[sub]],
                gbuf.at[pl.ds(sub * 128, 128)], sgat).wait()
        def _mul(t, _):
            for u in range(4):
                sl = pl.ds(t * 64 + u * 16, 16)
                gbuf[sl] = gbuf[sl] * sbuf[sl]
            return 0
        lax.fori_loop(0, _PB // 64, _mul, 0)
        pltpu.sync_copy(gbuf, xo_hbm.at[pl.ds(c * _KP + pbase, _PB)])

    def _start_in(b, cid):
        pltpu.async_copy(
            edge_hbm.at[pl.ds(0, 2), pl.ds(cid * _ECH, _ECH)], ein[b], sin[b])
    def _wait_in(b, cid):
        pltpu.make_async_copy(
            edge_hbm.at[pl.ds(0, 2), pl.ds(cid * _ECH, _ECH)], ein[b],
            sin[b]).wait()
    def _start_out(b, cid):
        pltpu.async_copy(
            eout_b[b], eout_hbm.at[pl.ds(0, 2), pl.ds(cid * _ECH, _ECH)],
            sout[b])
    def _wait_out(b, cid):
        pltpu.make_async_copy(
            eout_b[b], eout_hbm.at[pl.ds(0, 2), pl.ds(cid * _ECH, _ECH)],
            sout[b]).wait()

    _start_in(0, wid)
    _start_in(1, wid + _NW)

    def _epair(i, _):
        for b in range(2):
            cid = wid + (2 * i + b) * _NW

            @pl.when(cid < _NCHG)
            def _(b=b, cid=cid):
                _wait_in(b, cid)

            @pl.when((i > 0) & (cid - 2 * _NW < _NCHG))
            def _(b=b, cid=cid):
                _wait_out(b, cid - 2 * _NW)

            @pl.when(cid < _NCHG)
            def _(b=b, cid=cid):
                def _ebody(t, _, b=b):
                    for u in range(8):
                        sl = pl.ds(t * 128 + u * 16, 16)
                        rs = plsc.load_gather(node_map, [ein[b][0, sl]])
                        cs = plsc.load_gather(node_map, [ein[b][1, sl]])
                        keep = (rs >= 0) & (cs >= 0)
                        eout_b[b][0, sl] = jnp.where(keep, rs, -1)
                        eout_b[b][1, sl] = jnp.where(keep, cs, -1)
                    return 0
                lax.fori_loop(0, _ECH // 128, _ebody, 0)
                _start_out(b, cid)

            @pl.when(cid + 2 * _NW < _NCHG)
            def _(b=b, cid=cid):
                _start_in(b, cid + 2 * _NW)
        return 0
    lax.fori_loop(0, _SLOTS // 2, _epair, 0)
    for slot in (_SLOTS - 2, _SLOTS - 1):
        cid = wid + slot * _NW

        @pl.when(cid < _NCHG)
        def _(b=slot & 1, cid=cid):
            _wait_out(b, cid)


def _make_sc_remap():
    return functools.partial(
        pl.kernel,
    out_type=(
        jax.ShapeDtypeStruct((2, _E), jnp.int32),
        jax.ShapeDtypeStruct((_C * _KP,), jnp.float32),
    ),
    mesh=plsc.VectorSubcoreMesh(
        core_axis_name="c", subcore_axis_name="s",
        num_cores=_NC, num_subcores=_NS),
    compiler_params=pltpu.CompilerParams(needs_layout_passes=False),
    scratch_types=[
        pltpu.VMEM((_NMP,), jnp.int32),
        pltpu.VMEM((_PB,), jnp.int32),
        pltpu.VMEM((_PB,), jnp.float32),
        pltpu.VMEM((_PB // 128, 128), jnp.int32),
        pltpu.VMEM((_PB,), jnp.float32),
        pltpu.VMEM((_PCH,), jnp.int32),
        pltpu.VMEM((_PCH,), jnp.int32),
        pltpu.VMEM((2, _ECH), jnp.int32),
        pltpu.VMEM((2, _ECH), jnp.int32),
        pltpu.VMEM((2, _ECH), jnp.int32),
        pltpu.VMEM((2, _ECH), jnp.int32),
        pltpu.SemaphoreType.DMA,
        pltpu.SemaphoreType.DMA,
        pltpu.SemaphoreType.DMA,
        pltpu.SemaphoreType.DMA,
        pltpu.SemaphoreType.DMA,
    ],
    )(_sc_body)


def kernel(x, edge_index, weight):
    score = jnp.tanh(
        jnp.sum(x * weight[None, :], axis=-1)
        / (jnp.linalg.norm(weight) + 1e-16))
    spad = jnp.pad(score, (0, _SZ - _N), constant_values=-jnp.inf)
    sarr = spad.reshape(_L, _R).T
    ks, iz = _score_sort(sarr)
    score_sorted = ks.swapaxes(0, 1).reshape(_SZ)
    perm_sorted = iz.swapaxes(0, 1).reshape(_SZ)
    perm = perm_sorted[:_K]
    score_sel = score_sorted[:_K]

    perm_pad = jnp.pad(perm, (0, _KP - _K))
    score_pad = jnp.pad(score_sel, (0, _KP - _K))
    eout, xo = _make_sc_remap()(
        edge_index, perm_pad, score_pad, x.reshape(-1))
    x_out = xo.reshape(_C, _KP)[:, :_K].T
    return x_out, eout, perm
